# baseline (speedup 1.0000x reference)
import jax
import jax.numpy as jnp
from jax import lax
from jax.experimental import pallas as pl
from jax.experimental.pallas import tpu as pltpu
from jax.experimental.pallas import tpu_sc as plsc

VOCAB = 1000000
D = 64
B = 4096
T = 50

_info = plsc.get_sparse_core_info()
NC, NS = _info.num_cores, _info.num_subcores
NW = NC * NS

N_TOK = B * T
TOK_PER_W = N_TOK // NW
CHUNK = 200
GATHERS = TOK_PER_W // CHUNK
NBUF = 4
OUTER = GATHERS // NBUF


def _gather_kernel(table_hbm, idx_hbm, out_hbm, idx_v, rows_v, gsem, ssem):
    wid = lax.axis_index("s") * NC + lax.axis_index("c")
    base = wid * TOK_PER_W
    pltpu.sync_copy(idx_hbm.at[pl.ds(base, TOK_PER_W)], idx_v)

    def body(o, carry):
        j0 = o * NBUF
        gd = [
            pltpu.async_copy(
                table_hbm.at[idx_v.at[pl.ds((j0 + b) * CHUNK, CHUNK)]],
                rows_v.at[b],
                gsem.at[b],
            )
            for b in range(NBUF)
        ]
        sd = []
        for b in range(NBUF):
            gd[b].wait()
            sd.append(
                pltpu.async_copy(
                    rows_v.at[b],
                    out_hbm.at[pl.ds(base + (j0 + b) * CHUNK, CHUNK)],
                    ssem.at[b],
                )
            )
        for b in range(NBUF):
            sd[b].wait()
        return carry

    lax.fori_loop(0, OUTER, body, 0)


def _run(tokd, table2):
    mesh = plsc.VectorSubcoreMesh(core_axis_name="c", subcore_axis_name="s")
    k = pl.kernel(
        _gather_kernel,
        mesh=mesh,
        out_type=jax.ShapeDtypeStruct((N_TOK, 2 * D), jnp.float32),
        scratch_types=[
            pltpu.VMEM((TOK_PER_W,), jnp.int32),
            pltpu.VMEM((NBUF, CHUNK, 2 * D), jnp.float32),
            pltpu.SemaphoreType.DMA((NBUF,)),
            pltpu.SemaphoreType.DMA((NBUF,)),
        ],
        compiler_params=pltpu.CompilerParams(use_tc_tiling_on_sc=True),
    )
    return k(table2, tokd)


def kernel(tokens_or_embeds, embed_table):
    flat = tokens_or_embeds.reshape(N_TOK)
    tokd = flat // 2
    sel = (flat % 2).astype(bool)
    table2 = embed_table.reshape(VOCAB // 2, 2 * D)
    pairs = _run(tokd, table2)
    out = jnp.where(sel[:, None], pairs[:, D:], pairs[:, :D])
    return out.reshape(B, T, D)


# speedup vs baseline: 1.3930x; 1.3930x over previous
import jax
import jax.numpy as jnp
from jax import lax
from jax.experimental import pallas as pl
from jax.experimental.pallas import tpu as pltpu
from jax.experimental.pallas import tpu_sc as plsc

VOCAB = 1000000
D = 64
B = 4096
T = 50

_info = plsc.get_sparse_core_info()
NC, NS = _info.num_cores, _info.num_subcores
NW = NC * NS

ROWS_PER_W = B // NW
RPG = 1
GATHERS = ROWS_PER_W // RPG
NBUF = 8
OUTER = GATHERS // NBUF


def _gather_kernel(table_hbm, idx_hbm, out_hbm, idx_v, rows_v, gsem, ssem):
    wid = lax.axis_index("s") * NC + lax.axis_index("c")
    base = wid * ROWS_PER_W
    pltpu.sync_copy(idx_hbm.at[pl.ds(base, ROWS_PER_W)], idx_v)

    def body(o, carry):
        j0 = o * NBUF
        gd = [
            pltpu.async_copy(
                table_hbm.at[idx_v.at[j0 + b]],
                rows_v.at[b],
                gsem.at[b],
            )
            for b in range(NBUF)
        ]
        sd = []
        for b in range(NBUF):
            gd[b].wait()
            sd.append(
                pltpu.async_copy(
                    rows_v.at[b],
                    out_hbm.at[base + j0 + b, pl.ds(0, T), pl.ds(0, D)],
                    ssem.at[b],
                )
            )
        for b in range(NBUF):
            sd[b].wait()
        return carry

    lax.fori_loop(0, OUTER, body, 0)


def _run(tokens, embed_table):
    mesh = plsc.VectorSubcoreMesh(core_axis_name="c", subcore_axis_name="s")
    k = pl.kernel(
        _gather_kernel,
        mesh=mesh,
        out_type=jax.ShapeDtypeStruct((B, 56, 128), jnp.float32),
        scratch_types=[
            pltpu.VMEM((ROWS_PER_W, T), jnp.int32),
            pltpu.VMEM((NBUF, T, D), jnp.float32),
            pltpu.SemaphoreType.DMA((NBUF,)),
            pltpu.SemaphoreType.DMA((NBUF,)),
        ],
        compiler_params=pltpu.CompilerParams(use_tc_tiling_on_sc=False),
    )
    return k(embed_table, tokens)


def kernel(tokens_or_embeds, embed_table):
    out_padded = _run(tokens_or_embeds, embed_table)
    return out_padded[:, :T, :D]


# speedup vs baseline: 1.3936x; 1.0004x over previous
import jax
import jax.numpy as jnp
from jax import lax
from jax.experimental import pallas as pl
from jax.experimental.pallas import tpu as pltpu
from jax.experimental.pallas import tpu_sc as plsc

VOCAB = 1000000
D = 64
B = 4096
T = 50

_info = plsc.get_sparse_core_info()
NC, NS = _info.num_cores, _info.num_subcores
NW = NC * NS

ROWS_PER_W = B // NW
NBUF = 8
OUTER = ROWS_PER_W // NBUF


def _gather_kernel(table_hbm, idx_hbm, out_hbm, idx_v, rows_v, gsem, ssem):
    wid = lax.axis_index("s") * NC + lax.axis_index("c")
    base = wid * ROWS_PER_W
    pltpu.sync_copy(idx_hbm.at[pl.ds(base, ROWS_PER_W)], idx_v)

    def body(o, carry):
        j0 = o * NBUF
        gd = [
            pltpu.async_copy(
                table_hbm.at[idx_v.at[j0 + b]],
                rows_v.at[b],
                gsem.at[b],
            )
            for b in range(NBUF)
        ]
        sd = []
        for b in range(NBUF):
            gd[b].wait()
            sd.append(
                pltpu.async_copy(
                    rows_v.at[b],
                    out_hbm.at[base + j0 + b, pl.ds(0, T), pl.ds(0, D)],
                    ssem.at[b],
                )
            )
        for b in range(NBUF):
            sd[b].wait()
        return carry

    lax.fori_loop(0, OUTER, body, 0)


def _run(tokens, embed_table):
    mesh = plsc.VectorSubcoreMesh(core_axis_name="c", subcore_axis_name="s")
    k = pl.kernel(
        _gather_kernel,
        mesh=mesh,
        out_type=jax.ShapeDtypeStruct((B, 56, 128), jnp.float32),
        scratch_types=[
            pltpu.VMEM((ROWS_PER_W, T), jnp.int32),
            pltpu.VMEM((NBUF, T, D), jnp.float32),
            pltpu.SemaphoreType.DMA((NBUF,)),
            pltpu.SemaphoreType.DMA((NBUF,)),
        ],
        compiler_params=pltpu.CompilerParams(use_tc_tiling_on_sc=False),
    )
    return k(embed_table, tokens)


def kernel(tokens_or_embeds, embed_table):
    out_padded = _run(tokens_or_embeds, embed_table)
    return out_padded[:, :T, :D]
